# Initial kernel scaffold; baseline (speedup 1.0000x reference)
import jax
import jax.numpy as jnp
from jax.experimental import pallas as pl


def kernel(h, edge_index, norm, edge_weight, W, b):
    raise NotImplementedError("write your pallas kernel here")


# speedup vs baseline: 5.0466x; 5.0466x over previous
import functools

import jax
import jax.numpy as jnp
from jax import lax
from jax.experimental import pallas as pl
from jax.experimental.pallas import tpu as pltpu
from jax.experimental.pallas import tpu_sc as plsc

N, E, D_IN, D_OUT = 10000, 160000, 256, 256
H = D_OUT // 2
NTILES = 16
EDGES_PER_TILE = E // NTILES
KB = 80
NB = EDGES_PER_TILE // KB
RPT = 624
XBASE = NTILES * RPT
FCH = H // 16
PB = 208


def _mm_body(h_ref, w_ref, norm_ref, o_ref):
    hw = jax.lax.dot_general(
        h_ref[...], w_ref[...], (((1,), (0,)), ((), ())),
        precision=jax.lax.Precision.HIGHEST,
        preferred_element_type=jnp.float32)
    o_ref[...] = hw * norm_ref[...]


def _matmul_halves(h, W, norm):
    RB = 2000
    return pl.pallas_call(
        _mm_body,
        grid=(2, N // RB),
        in_specs=[
            pl.BlockSpec((RB, D_IN), lambda c, r: (r, 0)),
            pl.BlockSpec((D_IN, H), lambda c, r: (0, c)),
            pl.BlockSpec((RB, 1), lambda c, r: (r, 0)),
        ],
        out_specs=pl.BlockSpec((RB, H), lambda c, r: (c * (N // RB) + r, 0)),
        out_shape=jax.ShapeDtypeStruct((2 * N, H), jnp.float32),
    )(h, W, norm)


def _sc_body(hn_hbm, srcoff_hbm, dst_hbm, ew_hbm, norm_hbm, b_hbm, out_hbm,
             buf_v, src_v, dst_v, w_v, norm_v, b_v, agg_sh, sem):
    c = lax.axis_index("c")
    s = lax.axis_index("s")
    zbase = pl.multiple_of(s * RPT, 8)
    cH = pl.multiple_of(c * H, 8)

    zero16 = jnp.zeros((16,), jnp.float32)

    def _zero_row(i, carry):
        for q in range(FCH):
            buf_v[i, pl.ds(q * 16, 16)] = zero16
        return carry

    lax.fori_loop(0, PB, _zero_row, 0)
    for k in range(RPT // PB):
        pltpu.sync_copy(buf_v, agg_sh.at[pl.ds(zbase + k * PB, PB)])

    @pl.when(s < 2)
    def _zero_extra():
        xb = pl.multiple_of(XBASE + s * 8, 8)
        pltpu.sync_copy(buf_v.at[pl.ds(0, 8)], agg_sh.at[pl.ds(xb, 8)])

    plsc.subcore_barrier()

    gbuf = buf_v.at[pl.ds(0, KB)]

    def _batch(g, carry):
        base = pl.multiple_of(s * EDGES_PER_TILE + g * KB, 8)
        sbase = pl.multiple_of(c * E + s * EDGES_PER_TILE + g * KB, 8)
        pltpu.sync_copy(srcoff_hbm.at[pl.ds(sbase, KB)], src_v)
        pltpu.sync_copy(dst_hbm.at[pl.ds(base, KB)], dst_v)
        pltpu.sync_copy(ew_hbm.at[pl.ds(base, KB)], w_v)
        pltpu.async_copy(hn_hbm.at[src_v], gbuf, sem).wait()

        def _scale(g2, cc):
            wvec = w_v[pl.ds(pl.multiple_of(g2 * 16, 16), 16)]
            for l in range(16):
                wj = wvec[l]
                j = g2 * 16 + l
                for q in range(FCH):
                    sl = pl.ds(q * 16, 16)
                    buf_v[j, sl] = buf_v[j, sl] * wj
            return cc

        lax.fori_loop(0, KB // 16, _scale, 0)
        pltpu.sync_copy(gbuf, agg_sh.at[dst_v], add=True)
        return carry

    lax.fori_loop(0, NB, _batch, 0)
    plsc.subcore_barrier()

    pltpu.sync_copy(b_hbm.at[pl.ds(cH, H)], b_v)
    pltpu.sync_copy(norm_hbm.at[pl.ds(zbase, RPT)], norm_v)

    def _relu_row(i, nv):
        for q in range(FCH):
            sl = pl.ds(q * 16, 16)
            buf_v[i, sl] = jnp.maximum(buf_v[i, sl] * nv + b_v[sl], 0.0)

    for k in range(RPT // PB):
        nb = pl.multiple_of(zbase + k * PB, 8)
        pltpu.sync_copy(agg_sh.at[pl.ds(nb, PB)], buf_v)

        def _ep_group(g2, carry):
            nvec = norm_v[pl.ds(pl.multiple_of(k * PB + g2 * 16, 16), 16)]
            for l in range(16):
                _relu_row(g2 * 16 + l, nvec[l])
            return carry

        lax.fori_loop(0, PB // 16, _ep_group, 0)
        pltpu.sync_copy(buf_v, out_hbm.at[pl.ds(nb, PB), pl.ds(cH, H)])

    @pl.when(s < 2)
    def _finish_extra():
        xb = pl.multiple_of(XBASE + s * 8, 8)
        pltpu.sync_copy(agg_sh.at[pl.ds(xb, 8)], buf_v.at[pl.ds(0, 8)])
        pltpu.sync_copy(norm_hbm.at[pl.ds(xb, 8)], norm_v.at[pl.ds(0, 8)])
        nvec = norm_v[pl.ds(0, 16)]
        for l in range(8):
            _relu_row(l, nvec[l])
        pltpu.sync_copy(buf_v.at[pl.ds(0, 8)],
                        out_hbm.at[pl.ds(xb, 8), pl.ds(cH, H)])


_sc_aggregate = functools.partial(
    pl.kernel,
    out_type=jax.ShapeDtypeStruct((N, D_OUT), jnp.float32),
    mesh=plsc.VectorSubcoreMesh(core_axis_name="c", subcore_axis_name="s"),
    scratch_types=[
        pltpu.VMEM((PB, H), jnp.float32),
        pltpu.VMEM((KB,), jnp.int32),
        pltpu.VMEM((KB,), jnp.int32),
        pltpu.VMEM((KB,), jnp.float32),
        pltpu.VMEM((RPT,), jnp.float32),
        pltpu.VMEM((H,), jnp.float32),
        pltpu.VMEM_SHARED((N, H), jnp.float32),
        pltpu.SemaphoreType.DMA,
    ],
)(_sc_body)


def kernel(h, edge_index, norm, edge_weight, W, b):
    hn = _matmul_halves(h, W, norm)
    src = edge_index[0]
    srcoff = jnp.concatenate([src, src + N])
    return _sc_aggregate(hn, srcoff, edge_index[1], edge_weight,
                         norm.reshape(N), b)


# speedup vs baseline: 10.3572x; 2.0523x over previous
import functools

import jax
import jax.numpy as jnp
from jax import lax
from jax.experimental import pallas as pl
from jax.experimental.pallas import tpu as pltpu
from jax.experimental.pallas import tpu_sc as plsc

N, E, D_IN, D_OUT = 10000, 160000, 256, 256
H = D_OUT // 2
NTILES = 16
EPT = E // NTILES
KB = 80
NBT = EPT // KB
NPAIR = (NBT - 1) // 2
RPT = 624
XBASE = NTILES * RPT
FCH = H // 16
ZB = 80


def _mm_body(h_ref, w_ref, norm_ref, o_ref):
    hw = jax.lax.dot_general(
        h_ref[...], w_ref[...], (((1,), (0,)), ((), ())),
        precision=jax.lax.Precision.HIGHEST,
        preferred_element_type=jnp.float32)
    o_ref[...] = hw * norm_ref[...]


def _matmul_halves(h, W, norm):
    RB = 2000
    return pl.pallas_call(
        _mm_body,
        grid=(2, N // RB),
        in_specs=[
            pl.BlockSpec((RB, D_IN), lambda c, r: (r, 0)),
            pl.BlockSpec((D_IN, H), lambda c, r: (0, c)),
            pl.BlockSpec((RB, 1), lambda c, r: (r, 0)),
        ],
        out_specs=pl.BlockSpec((RB, H), lambda c, r: (c * (N // RB) + r, 0)),
        out_shape=jax.ShapeDtypeStruct((2 * N, H), jnp.float32),
    )(h, W, norm)


def _sc_body(hn_hbm, srcoff_hbm, dst_hbm, ew_hbm, norm_hbm, b_hbm, out_hbm,
             gbuf0, gbuf1, srcC, wC, dstv0, dstv1, norm_v, b_v,
             agg_sh, semg0, semg1, sems0, sems1, semE):
    c = lax.axis_index("c")
    s = lax.axis_index("s")
    zbase = pl.multiple_of(s * RPT, 8)
    cH = pl.multiple_of(c * H, 8)
    ebase = pl.multiple_of(s * EPT, 8)
    zero16 = jnp.zeros((16,), jnp.float32)

    esbase = pl.multiple_of(c * E + s * EPT, 8)
    d_src = pltpu.async_copy(srcoff_hbm.at[pl.ds(esbase, EPT)], srcC, semE)
    d_w = pltpu.async_copy(ew_hbm.at[pl.ds(ebase, EPT)], wC, semE)

    def _zero_row(i, carry):
        for q in range(FCH):
            gbuf0[i, pl.ds(q * 16, 16)] = zero16
        return carry

    lax.fori_loop(0, ZB, _zero_row, 0)

    def _zero_block(k, carry):
        pltpu.sync_copy(gbuf0, agg_sh.at[pl.ds(zbase + k * ZB, ZB)])
        return carry

    lax.fori_loop(0, RPT // ZB, _zero_block, 0)
    pltpu.sync_copy(gbuf0.at[pl.ds(0, 64)],
                    agg_sh.at[pl.ds(zbase + (RPT // ZB) * ZB, 64)])

    @pl.when(s < 2)
    def _zero_extra():
        xb = pl.multiple_of(XBASE + s * 8, 8)
        pltpu.sync_copy(gbuf0.at[pl.ds(0, 8)], agg_sh.at[pl.ds(xb, 8)])

    d_src.wait()
    d_w.wait()
    plsc.subcore_barrier()

    def _src_slice(j):
        return srcC.at[pl.ds(pl.multiple_of(j * KB, 16), KB)]

    def _fetch_start(gb, dv, j, sem):
        pltpu.async_copy(
            dst_hbm.at[pl.ds(pl.multiple_of(ebase + j * KB, 8), KB)], dv, sem)
        pltpu.async_copy(hn_hbm.at[_src_slice(j)], gb, sem)

    def _fetch_wait(gb, dv, sem):
        pltpu.make_async_copy(dst_hbm.at[pl.ds(ebase, KB)], dv, sem).wait()
        pltpu.make_async_copy(hn_hbm.at[_src_slice(0)], gb, sem).wait()

    def _sstart(gb, dv, sem):
        pltpu.make_async_copy(gb, agg_sh.at[dv], sem).start(add=True)

    def _swait(gb, dv, sem):
        pltpu.make_async_copy(gb, agg_sh.at[dv], sem).wait()

    def _scale(gb, j):
        def _grp(g2, cc):
            wvec = wC[pl.ds(pl.multiple_of(j * KB + g2 * 16, 16), 16)]
            for l in range(16):
                wj = wvec[l]
                r = g2 * 16 + l
                for q in range(FCH):
                    sl = pl.ds(q * 16, 16)
                    gb[r, sl] = gb[r, sl] * wj
            return cc

        lax.fori_loop(0, KB // 16, _grp, 0)

    _fetch_start(gbuf0, dstv0, 0, semg0)
    _fetch_start(gbuf1, dstv1, 1, semg1)

    def _pair(p, carry):
        a = 2 * p
        _fetch_wait(gbuf0, dstv0, semg0)
        _scale(gbuf0, a)
        _sstart(gbuf0, dstv0, sems0)
        _fetch_wait(gbuf1, dstv1, semg1)
        _scale(gbuf1, a + 1)
        _sstart(gbuf1, dstv1, sems1)
        na = jnp.minimum(a + 2, NBT - 1)
        nb = jnp.minimum(a + 3, NBT - 1)
        _swait(gbuf0, dstv0, sems0)
        _fetch_start(gbuf0, dstv0, na, semg0)
        _swait(gbuf1, dstv1, sems1)
        _fetch_start(gbuf1, dstv1, nb, semg1)
        return carry

    lax.fori_loop(0, NPAIR, _pair, 0)
    _fetch_wait(gbuf0, dstv0, semg0)
    _scale(gbuf0, NBT - 1)
    _sstart(gbuf0, dstv0, sems0)
    _fetch_wait(gbuf1, dstv1, semg1)
    _swait(gbuf0, dstv0, sems0)
    plsc.subcore_barrier()

    pltpu.sync_copy(b_hbm.at[pl.ds(cH, H)], b_v)

    def _relu_rows(ngroups):
        def _eg(g2, carry):
            nvec = norm_v[pl.ds(pl.multiple_of(g2 * 16, 16), 16)]
            for l in range(16):
                r = g2 * 16 + l
                for q in range(FCH):
                    sl = pl.ds(q * 16, 16)
                    gbuf0[r, sl] = jnp.maximum(
                        gbuf0[r, sl] * nvec[l] + b_v[sl], 0.0)
            return carry

        lax.fori_loop(0, ngroups, _eg, 0)

    def _ep_block(k, carry):
        off = pl.multiple_of(zbase + k * ZB, 8)
        pltpu.sync_copy(agg_sh.at[pl.ds(off, ZB)], gbuf0)
        pltpu.sync_copy(norm_hbm.at[pl.ds(off, ZB)], norm_v.at[pl.ds(0, ZB)])
        _relu_rows(ZB // 16)
        pltpu.sync_copy(gbuf0, out_hbm.at[pl.ds(off, ZB), pl.ds(cH, H)])
        return carry

    lax.fori_loop(0, RPT // ZB, _ep_block, 0)

    offp = pl.multiple_of(zbase + (RPT // ZB) * ZB, 8)
    pltpu.sync_copy(agg_sh.at[pl.ds(offp, 64)], gbuf0.at[pl.ds(0, 64)])
    pltpu.sync_copy(norm_hbm.at[pl.ds(offp, 64)], norm_v.at[pl.ds(0, 64)])
    _relu_rows(4)
    pltpu.sync_copy(gbuf0.at[pl.ds(0, 64)],
                    out_hbm.at[pl.ds(offp, 64), pl.ds(cH, H)])

    @pl.when(s < 2)
    def _finish_extra():
        xb = pl.multiple_of(XBASE + s * 8, 8)
        pltpu.sync_copy(agg_sh.at[pl.ds(xb, 8)], gbuf0.at[pl.ds(0, 8)])
        pltpu.sync_copy(norm_hbm.at[pl.ds(xb, 8)], norm_v.at[pl.ds(0, 8)])
        nvec = norm_v[pl.ds(0, 16)]
        for l in range(8):
            for q in range(FCH):
                sl = pl.ds(q * 16, 16)
                gbuf0[l, sl] = jnp.maximum(
                    gbuf0[l, sl] * nvec[l] + b_v[sl], 0.0)
        pltpu.sync_copy(gbuf0.at[pl.ds(0, 8)],
                        out_hbm.at[pl.ds(xb, 8), pl.ds(cH, H)])


_sc_aggregate = functools.partial(
    pl.kernel,
    out_type=jax.ShapeDtypeStruct((N, D_OUT), jnp.float32),
    mesh=plsc.VectorSubcoreMesh(core_axis_name="c", subcore_axis_name="s"),
    scratch_types=[
        pltpu.VMEM((KB, H), jnp.float32),
        pltpu.VMEM((KB, H), jnp.float32),
        pltpu.VMEM((EPT,), jnp.int32),
        pltpu.VMEM((EPT,), jnp.float32),
        pltpu.VMEM((KB,), jnp.int32),
        pltpu.VMEM((KB,), jnp.int32),
        pltpu.VMEM((ZB,), jnp.float32),
        pltpu.VMEM((H,), jnp.float32),
        pltpu.VMEM_SHARED((N, H), jnp.float32),
        pltpu.SemaphoreType.DMA,
        pltpu.SemaphoreType.DMA,
        pltpu.SemaphoreType.DMA,
        pltpu.SemaphoreType.DMA,
        pltpu.SemaphoreType.DMA,
    ],
)(_sc_body)


def kernel(h, edge_index, norm, edge_weight, W, b):
    hn = _matmul_halves(h, W, norm)
    src = edge_index[0]
    srcoff = jnp.concatenate([src, src + N])
    return _sc_aggregate(hn, srcoff, edge_index[1], edge_weight,
                         norm.reshape(N), b)


# speedup vs baseline: 11.2149x; 1.0828x over previous
import functools

import jax
import jax.numpy as jnp
from jax import lax
from jax.experimental import pallas as pl
from jax.experimental.pallas import tpu as pltpu
from jax.experimental.pallas import tpu_sc as plsc

N, E, D_IN, D_OUT = 10000, 160000, 256, 256
H = D_OUT // 2
NTILES = 16
EPT = E // NTILES
KB = 80
NBT = EPT // KB
NSLOT = 3
NTRI = (NBT - 2) // NSLOT
RPT = 624
XBASE = NTILES * RPT
FCH = H // 16
ZB = 80


def _mm_body(h_ref, w_ref, norm_ref, o_ref):
    hw = jax.lax.dot_general(
        h_ref[...], w_ref[...], (((1,), (0,)), ((), ())),
        precision=jax.lax.Precision.HIGHEST,
        preferred_element_type=jnp.float32)
    o_ref[...] = hw * norm_ref[...]


def _matmul_halves(h, W, norm):
    RB = 2000
    return pl.pallas_call(
        _mm_body,
        grid=(2, N // RB),
        in_specs=[
            pl.BlockSpec((RB, D_IN), lambda c, r: (r, 0)),
            pl.BlockSpec((D_IN, H), lambda c, r: (0, c)),
            pl.BlockSpec((RB, 1), lambda c, r: (r, 0)),
        ],
        out_specs=pl.BlockSpec((RB, H), lambda c, r: (c * (N // RB) + r, 0)),
        out_shape=jax.ShapeDtypeStruct((2 * N, H), jnp.float32),
    )(h, W, norm)


def _sc_body(hn_hbm, ei_hbm, ew_hbm, norm_hbm, b_hbm, out_hbm,
             gbuf0, gbuf1, gbuf2, srcC, dstv0, dstv1, dstv2,
             wv0, wv1, wv2, norm_v, b_v, agg_sh,
             semg0, semg1, semg2, sems0, sems1, sems2, semE):
    c = lax.axis_index("c")
    s = lax.axis_index("s")
    zbase = pl.multiple_of(s * RPT, 8)
    cH = pl.multiple_of(c * H, 8)
    ebase = pl.multiple_of(s * EPT, 8)
    zero16 = jnp.zeros((16,), jnp.float32)

    d_src = pltpu.async_copy(ei_hbm.at[pl.ds(ebase, EPT)], srcC, semE)

    def _zero_row(i, carry):
        for q in range(FCH):
            gbuf0[i, pl.ds(q * 16, 16)] = zero16
        return carry

    lax.fori_loop(0, ZB, _zero_row, 0)

    def _zero_block(k, carry):
        pltpu.sync_copy(gbuf0, agg_sh.at[pl.ds(zbase + k * ZB, ZB)])
        return carry

    lax.fori_loop(0, RPT // ZB, _zero_block, 0)
    pltpu.sync_copy(gbuf0.at[pl.ds(0, 64)],
                    agg_sh.at[pl.ds(zbase + (RPT // ZB) * ZB, 64)])

    @pl.when(s < 2)
    def _zero_extra():
        xb = pl.multiple_of(XBASE + s * 8, 8)
        pltpu.sync_copy(gbuf0.at[pl.ds(0, 8)], agg_sh.at[pl.ds(xb, 8)])

    d_src.wait()
    cN = jnp.full((16,), c * N, jnp.int32)

    def _off_row(i, carry):
        sl = pl.ds(pl.multiple_of(i * 16, 16), 16)
        srcC[sl] = srcC[sl] + cN
        return carry

    lax.fori_loop(0, EPT // 16, _off_row, 0)
    plsc.subcore_barrier()

    def _fetch_start(gb, dv, wv, j, sem):
        pltpu.async_copy(
            ei_hbm.at[pl.ds(pl.multiple_of(E + ebase + j * KB, 8), KB)],
            dv, sem)
        pltpu.async_copy(
            ew_hbm.at[pl.ds(pl.multiple_of(ebase + j * KB, 8), KB)], wv, sem)
        pltpu.async_copy(
            hn_hbm.at[srcC.at[pl.ds(pl.multiple_of(j * KB, 16), KB)]],
            gb, sem)

    def _fetch_wait(gb, dv, wv, sem):
        pltpu.make_async_copy(ei_hbm.at[pl.ds(E, KB)], dv, sem).wait()
        pltpu.make_async_copy(ew_hbm.at[pl.ds(0, KB)], wv, sem).wait()
        pltpu.make_async_copy(
            hn_hbm.at[srcC.at[pl.ds(0, KB)]], gb, sem).wait()

    def _sstart(gb, dv, sem):
        pltpu.make_async_copy(gb, agg_sh.at[dv], sem).start(add=True)

    def _swait(gb, dv, sem):
        pltpu.make_async_copy(gb, agg_sh.at[dv], sem).wait()

    def _scale(gb, wv):
        def _grp(g2, cc):
            wvec = wv[pl.ds(pl.multiple_of(g2 * 16, 16), 16)]
            for l in range(16):
                wj = wvec[l]
                r = g2 * 16 + l
                for q in range(FCH):
                    sl = pl.ds(q * 16, 16)
                    gb[r, sl] = gb[r, sl] * wj
            return cc

        lax.fori_loop(0, KB // 16, _grp, 0)

    slots = ((gbuf0, dstv0, wv0, semg0, sems0),
             (gbuf1, dstv1, wv1, semg1, sems1),
             (gbuf2, dstv2, wv2, semg2, sems2))
    for k, (gb, dv, wv, sg, ss) in enumerate(slots):
        _fetch_start(gb, dv, wv, k, sg)

    def _triple(p, carry):
        a = NSLOT * p
        for k, (gb, dv, wv, sg, ss) in enumerate(slots):
            _fetch_wait(gb, dv, wv, sg)
            _scale(gb, wv)
            _sstart(gb, dv, ss)
        for k, (gb, dv, wv, sg, ss) in enumerate(slots):
            _swait(gb, dv, ss)
            nj = jnp.minimum(a + NSLOT + k, NBT - 1)
            _fetch_start(gb, dv, wv, nj, sg)
        return carry

    lax.fori_loop(0, NTRI, _triple, 0)
    for k in range(2):
        gb, dv, wv, sg, ss = slots[k]
        _fetch_wait(gb, dv, wv, sg)
        _scale(gb, wv)
        _sstart(gb, dv, ss)
    _fetch_wait(*slots[2][:3], slots[2][3])
    for k in range(2):
        gb, dv, wv, sg, ss = slots[k]
        _swait(gb, dv, ss)
    plsc.subcore_barrier()

    pltpu.sync_copy(b_hbm.at[pl.ds(cH, H)], b_v)

    def _relu_rows(ngroups):
        def _eg(g2, carry):
            nvec = norm_v[pl.ds(pl.multiple_of(g2 * 16, 16), 16)]
            for l in range(16):
                r = g2 * 16 + l
                for q in range(FCH):
                    sl = pl.ds(q * 16, 16)
                    gbuf0[r, sl] = jnp.maximum(
                        gbuf0[r, sl] * nvec[l] + b_v[sl], 0.0)
            return carry

        lax.fori_loop(0, ngroups, _eg, 0)

    def _ep_block(k, carry):
        off = pl.multiple_of(zbase + k * ZB, 8)
        pltpu.sync_copy(agg_sh.at[pl.ds(off, ZB)], gbuf0)
        pltpu.sync_copy(norm_hbm.at[pl.ds(off, ZB)], norm_v.at[pl.ds(0, ZB)])
        _relu_rows(ZB // 16)
        pltpu.sync_copy(gbuf0, out_hbm.at[pl.ds(off, ZB), pl.ds(cH, H)])
        return carry

    lax.fori_loop(0, RPT // ZB, _ep_block, 0)

    offp = pl.multiple_of(zbase + (RPT // ZB) * ZB, 8)
    pltpu.sync_copy(agg_sh.at[pl.ds(offp, 64)], gbuf0.at[pl.ds(0, 64)])
    pltpu.sync_copy(norm_hbm.at[pl.ds(offp, 64)], norm_v.at[pl.ds(0, 64)])
    _relu_rows(4)
    pltpu.sync_copy(gbuf0.at[pl.ds(0, 64)],
                    out_hbm.at[pl.ds(offp, 64), pl.ds(cH, H)])

    @pl.when(s < 2)
    def _finish_extra():
        xb = pl.multiple_of(XBASE + s * 8, 8)
        pltpu.sync_copy(agg_sh.at[pl.ds(xb, 8)], gbuf0.at[pl.ds(0, 8)])
        pltpu.sync_copy(norm_hbm.at[pl.ds(xb, 8)], norm_v.at[pl.ds(0, 8)])
        nvec = norm_v[pl.ds(0, 16)]
        for l in range(8):
            for q in range(FCH):
                sl = pl.ds(q * 16, 16)
                gbuf0[l, sl] = jnp.maximum(
                    gbuf0[l, sl] * nvec[l] + b_v[sl], 0.0)
        pltpu.sync_copy(gbuf0.at[pl.ds(0, 8)],
                        out_hbm.at[pl.ds(xb, 8), pl.ds(cH, H)])


_sc_aggregate = functools.partial(
    pl.kernel,
    out_type=jax.ShapeDtypeStruct((N, D_OUT), jnp.float32),
    mesh=plsc.VectorSubcoreMesh(core_axis_name="c", subcore_axis_name="s"),
    scratch_types=[
        pltpu.VMEM((KB, H), jnp.float32),
        pltpu.VMEM((KB, H), jnp.float32),
        pltpu.VMEM((KB, H), jnp.float32),
        pltpu.VMEM((EPT,), jnp.int32),
        pltpu.VMEM((KB,), jnp.int32),
        pltpu.VMEM((KB,), jnp.int32),
        pltpu.VMEM((KB,), jnp.int32),
        pltpu.VMEM((KB,), jnp.float32),
        pltpu.VMEM((KB,), jnp.float32),
        pltpu.VMEM((KB,), jnp.float32),
        pltpu.VMEM((ZB,), jnp.float32),
        pltpu.VMEM((H,), jnp.float32),
        pltpu.VMEM_SHARED((N, H), jnp.float32),
        pltpu.SemaphoreType.DMA,
        pltpu.SemaphoreType.DMA,
        pltpu.SemaphoreType.DMA,
        pltpu.SemaphoreType.DMA,
        pltpu.SemaphoreType.DMA,
        pltpu.SemaphoreType.DMA,
        pltpu.SemaphoreType.DMA,
    ],
)(_sc_body)


def kernel(h, edge_index, norm, edge_weight, W, b):
    hn = _matmul_halves(h, W, norm)
    return _sc_aggregate(hn, edge_index.reshape(2 * E), edge_weight,
                         norm.reshape(N), b)


# speedup vs baseline: 11.6685x; 1.0405x over previous
import functools

import jax
import jax.numpy as jnp
from jax import lax
from jax.experimental import pallas as pl
from jax.experimental.pallas import tpu as pltpu
from jax.experimental.pallas import tpu_sc as plsc

N, E, D_IN, D_OUT = 10000, 160000, 256, 256
H = D_OUT // 2
NTILES = 16
EPT = E // NTILES
KB = 80
NBT = EPT // KB
NSLOT = 3
NTRI = (NBT - 2) // NSLOT
RPT = 624
XBASE = NTILES * RPT
FCH = H // 16
ZB = 80


def _mm_body(h_ref, w_ref, norm_ref, o_ref):
    hw = jax.lax.dot_general(
        h_ref[...], w_ref[...], (((1,), (0,)), ((), ())),
        preferred_element_type=jnp.float32)
    o_ref[...] = hw * norm_ref[...]


def _matmul_halves(h, W, norm):
    RB = 2000
    return pl.pallas_call(
        _mm_body,
        grid=(2, N // RB),
        in_specs=[
            pl.BlockSpec((RB, D_IN), lambda c, r: (r, 0)),
            pl.BlockSpec((D_IN, H), lambda c, r: (0, c)),
            pl.BlockSpec((RB, 1), lambda c, r: (r, 0)),
        ],
        out_specs=pl.BlockSpec((RB, H), lambda c, r: (c * (N // RB) + r, 0)),
        out_shape=jax.ShapeDtypeStruct((2 * N, H), jnp.float32),
    )(h, W, norm)


def _sc_body(hn_hbm, ei_hbm, ew_hbm, norm_hbm, b_hbm, out_hbm,
             gbuf0, gbuf1, gbuf2, srcC, dstv0, dstv1, dstv2,
             wv0, wv1, wv2, norm_v, b_v, zimg, agg_sh,
             semg0, semg1, semg2, sems0, sems1, sems2, semE):
    c = lax.axis_index("c")
    s = lax.axis_index("s")
    zbase = pl.multiple_of(s * RPT, 8)
    cH = pl.multiple_of(c * H, 8)
    ebase = pl.multiple_of(s * EPT, 8)
    zero16 = jnp.zeros((16,), jnp.float32)

    pltpu.sync_copy(ei_hbm.at[pl.ds(ebase, EPT)], srcC)
    cN = jnp.full((16,), c * N, jnp.int32)

    def _off_row(i, carry):
        sl = pl.ds(pl.multiple_of(i * 16, 16), 16)
        srcC[sl] = srcC[sl] + cN
        return carry

    lax.fori_loop(0, EPT // 16, _off_row, 0)

    def _fetch_start(gb, dv, wv, j, sem):
        pltpu.async_copy(
            ei_hbm.at[pl.ds(pl.multiple_of(E + ebase + j * KB, 8), KB)],
            dv, sem)
        pltpu.async_copy(
            ew_hbm.at[pl.ds(pl.multiple_of(ebase + j * KB, 8), KB)], wv, sem)
        pltpu.async_copy(
            hn_hbm.at[srcC.at[pl.ds(pl.multiple_of(j * KB, 16), KB)]],
            gb, sem)

    def _fetch_wait(gb, dv, wv, sem):
        pltpu.make_async_copy(ei_hbm.at[pl.ds(E, KB)], dv, sem).wait()
        pltpu.make_async_copy(ew_hbm.at[pl.ds(0, KB)], wv, sem).wait()
        pltpu.make_async_copy(
            hn_hbm.at[srcC.at[pl.ds(0, KB)]], gb, sem).wait()

    def _sstart(gb, dv, sem):
        pltpu.make_async_copy(gb, agg_sh.at[dv], sem).start(add=True)

    def _swait(gb, dv, sem):
        pltpu.make_async_copy(gb, agg_sh.at[dv], sem).wait()

    def _scale(gb, wv):
        def _grp(g2, cc):
            wvec = wv[pl.ds(pl.multiple_of(g2 * 16, 16), 16)]
            for l in range(16):
                wj = wvec[l]
                r = g2 * 16 + l
                for q in range(FCH):
                    sl = pl.ds(q * 16, 16)
                    gb[r, sl] = gb[r, sl] * wj
            return cc

        lax.fori_loop(0, KB // 16, _grp, 0)

    slots = ((gbuf0, dstv0, wv0, semg0, sems0),
             (gbuf1, dstv1, wv1, semg1, sems1),
             (gbuf2, dstv2, wv2, semg2, sems2))
    for k, (gb, dv, wv, sg, ss) in enumerate(slots):
        _fetch_start(gb, dv, wv, k, sg)

    def _zimg_row(i, carry):
        for q in range(FCH):
            zimg[i, pl.ds(q * 16, 16)] = zero16
        return carry

    lax.fori_loop(0, 16, _zimg_row, 0)

    def _zfire(i, carry):
        pltpu.async_copy(zimg, agg_sh.at[pl.ds(zbase + i * 16, 16)], semE)
        return carry

    lax.fori_loop(0, RPT // 16, _zfire, 0)

    @pl.when(s < 2)
    def _zero_extra():
        xb = pl.multiple_of(XBASE + s * 8, 8)
        pltpu.async_copy(zimg.at[pl.ds(0, 8)], agg_sh.at[pl.ds(xb, 8)], semE)

    def _zdrain(i, carry):
        pltpu.make_async_copy(zimg, agg_sh.at[pl.ds(zbase, 16)], semE).wait()
        return carry

    lax.fori_loop(0, RPT // 16, _zdrain, 0)

    @pl.when(s < 2)
    def _zdrain_extra():
        pltpu.make_async_copy(zimg.at[pl.ds(0, 8)],
                              agg_sh.at[pl.ds(zbase, 8)], semE).wait()

    plsc.subcore_barrier()

    def _triple(p, carry):
        a = NSLOT * p
        for k, (gb, dv, wv, sg, ss) in enumerate(slots):
            _fetch_wait(gb, dv, wv, sg)
            _scale(gb, wv)
            _sstart(gb, dv, ss)
        for k, (gb, dv, wv, sg, ss) in enumerate(slots):
            _swait(gb, dv, ss)
            nj = jnp.minimum(a + NSLOT + k, NBT - 1)
            _fetch_start(gb, dv, wv, nj, sg)
        return carry

    lax.fori_loop(0, NTRI, _triple, 0)
    for k in range(2):
        gb, dv, wv, sg, ss = slots[k]
        _fetch_wait(gb, dv, wv, sg)
        _scale(gb, wv)
        _sstart(gb, dv, ss)
    _fetch_wait(slots[2][0], slots[2][1], slots[2][2], slots[2][3])
    for k in range(2):
        gb, dv, wv, sg, ss = slots[k]
        _swait(gb, dv, ss)
    plsc.subcore_barrier()

    pltpu.sync_copy(b_hbm.at[pl.ds(cH, H)], b_v)

    def _relu_rows(ngroups):
        def _eg(g2, carry):
            nvec = norm_v[pl.ds(pl.multiple_of(g2 * 16, 16), 16)]
            for l in range(16):
                r = g2 * 16 + l
                for q in range(FCH):
                    sl = pl.ds(q * 16, 16)
                    gbuf0[r, sl] = jnp.maximum(
                        gbuf0[r, sl] * nvec[l] + b_v[sl], 0.0)
            return carry

        lax.fori_loop(0, ngroups, _eg, 0)

    def _ep_block(k, carry):
        off = pl.multiple_of(zbase + k * ZB, 8)
        pltpu.sync_copy(agg_sh.at[pl.ds(off, ZB)], gbuf0)
        pltpu.sync_copy(norm_hbm.at[pl.ds(off, ZB)], norm_v.at[pl.ds(0, ZB)])
        _relu_rows(ZB // 16)
        pltpu.sync_copy(gbuf0, out_hbm.at[pl.ds(off, ZB), pl.ds(cH, H)])
        return carry

    lax.fori_loop(0, RPT // ZB, _ep_block, 0)

    offp = pl.multiple_of(zbase + (RPT // ZB) * ZB, 8)
    pltpu.sync_copy(agg_sh.at[pl.ds(offp, 64)], gbuf0.at[pl.ds(0, 64)])
    pltpu.sync_copy(norm_hbm.at[pl.ds(offp, 64)], norm_v.at[pl.ds(0, 64)])
    _relu_rows(4)
    pltpu.sync_copy(gbuf0.at[pl.ds(0, 64)],
                    out_hbm.at[pl.ds(offp, 64), pl.ds(cH, H)])

    @pl.when(s < 2)
    def _finish_extra():
        xb = pl.multiple_of(XBASE + s * 8, 8)
        pltpu.sync_copy(agg_sh.at[pl.ds(xb, 8)], gbuf0.at[pl.ds(0, 8)])
        pltpu.sync_copy(norm_hbm.at[pl.ds(xb, 8)], norm_v.at[pl.ds(0, 8)])
        nvec = norm_v[pl.ds(0, 16)]
        for l in range(8):
            for q in range(FCH):
                sl = pl.ds(q * 16, 16)
                gbuf0[l, sl] = jnp.maximum(
                    gbuf0[l, sl] * nvec[l] + b_v[sl], 0.0)
        pltpu.sync_copy(gbuf0.at[pl.ds(0, 8)],
                        out_hbm.at[pl.ds(xb, 8), pl.ds(cH, H)])


_sc_aggregate = functools.partial(
    pl.kernel,
    out_type=jax.ShapeDtypeStruct((N, D_OUT), jnp.float32),
    mesh=plsc.VectorSubcoreMesh(core_axis_name="c", subcore_axis_name="s"),
    scratch_types=[
        pltpu.VMEM((KB, H), jnp.float32),
        pltpu.VMEM((KB, H), jnp.float32),
        pltpu.VMEM((KB, H), jnp.float32),
        pltpu.VMEM((EPT,), jnp.int32),
        pltpu.VMEM((KB,), jnp.int32),
        pltpu.VMEM((KB,), jnp.int32),
        pltpu.VMEM((KB,), jnp.int32),
        pltpu.VMEM((KB,), jnp.float32),
        pltpu.VMEM((KB,), jnp.float32),
        pltpu.VMEM((KB,), jnp.float32),
        pltpu.VMEM((ZB,), jnp.float32),
        pltpu.VMEM((H,), jnp.float32),
        pltpu.VMEM((16, H), jnp.float32),
        pltpu.VMEM_SHARED((N, H), jnp.float32),
        pltpu.SemaphoreType.DMA,
        pltpu.SemaphoreType.DMA,
        pltpu.SemaphoreType.DMA,
        pltpu.SemaphoreType.DMA,
        pltpu.SemaphoreType.DMA,
        pltpu.SemaphoreType.DMA,
        pltpu.SemaphoreType.DMA,
    ],
)(_sc_body)


def kernel(h, edge_index, norm, edge_weight, W, b):
    hn = _matmul_halves(h, W, norm)
    return _sc_aggregate(hn, edge_index.reshape(2 * E), edge_weight,
                         norm.reshape(N), b)


# speedup vs baseline: 13.1364x; 1.1258x over previous
import functools

import jax
import jax.numpy as jnp
from jax import lax
from jax.experimental import pallas as pl
from jax.experimental.pallas import tpu as pltpu
from jax.experimental.pallas import tpu_sc as plsc

N, E, D_IN, D_OUT = 10000, 160000, 256, 256
H = D_OUT // 2
NTILES = 16
EPT = E // NTILES
KB = 80
NBT = EPT // KB
NSLOT = 3
NTRI = (NBT - 2) // NSLOT
RPT = 624
XBASE = NTILES * RPT
FCH = H // 16
ZB = 80


def _mm_body(h_ref, w_ref, norm_ref, o_ref):
    hw = jax.lax.dot_general(
        h_ref[...], w_ref[...], (((1,), (0,)), ((), ())),
        preferred_element_type=jnp.float32)
    o_ref[...] = hw * norm_ref[...]


def _matmul_halves(h, W, norm):
    RB = 2000
    return pl.pallas_call(
        _mm_body,
        grid=(2, N // RB),
        in_specs=[
            pl.BlockSpec((RB, D_IN), lambda c, r: (r, 0)),
            pl.BlockSpec((D_IN, H), lambda c, r: (0, c)),
            pl.BlockSpec((RB, 1), lambda c, r: (r, 0)),
        ],
        out_specs=pl.BlockSpec((RB, H), lambda c, r: (c * (N // RB) + r, 0)),
        out_shape=jax.ShapeDtypeStruct((2 * N, H), jnp.float32),
    )(h, W, norm)


def _sc_body(hn_hbm, ei_hbm, ew_hbm, out_hbm,
             gbuf0, gbuf1, gbuf2, srcC, dstv0, dstv1, dstv2,
             wv0, wv1, wv2, zimg, agg_sh,
             semg0, semg1, semg2, sems0, sems1, sems2, semE):
    c = lax.axis_index("c")
    s = lax.axis_index("s")
    zbase = pl.multiple_of(s * RPT, 8)
    cH = pl.multiple_of(c * H, 8)
    ebase = pl.multiple_of(s * EPT, 8)
    zero16 = jnp.zeros((16,), jnp.float32)

    pltpu.sync_copy(ei_hbm.at[pl.ds(ebase, EPT)], srcC)
    cN = jnp.full((16,), c * N, jnp.int32)

    def _off_row(i, carry):
        sl = pl.ds(pl.multiple_of(i * 16, 16), 16)
        srcC[sl] = srcC[sl] + cN
        return carry

    lax.fori_loop(0, EPT // 16, _off_row, 0)

    def _fetch_start(gb, dv, wv, j, sem):
        pltpu.async_copy(
            ei_hbm.at[pl.ds(pl.multiple_of(E + ebase + j * KB, 8), KB)],
            dv, sem)
        pltpu.async_copy(
            ew_hbm.at[pl.ds(pl.multiple_of(ebase + j * KB, 8), KB)], wv, sem)
        pltpu.async_copy(
            hn_hbm.at[srcC.at[pl.ds(pl.multiple_of(j * KB, 16), KB)]],
            gb, sem)

    def _fetch_wait(gb, dv, wv, sem):
        pltpu.make_async_copy(ei_hbm.at[pl.ds(E, KB)], dv, sem).wait()
        pltpu.make_async_copy(ew_hbm.at[pl.ds(0, KB)], wv, sem).wait()
        pltpu.make_async_copy(
            hn_hbm.at[srcC.at[pl.ds(0, KB)]], gb, sem).wait()

    def _sstart(gb, dv, sem):
        pltpu.make_async_copy(gb, agg_sh.at[dv], sem).start(add=True)

    def _swait(gb, dv, sem):
        pltpu.make_async_copy(gb, agg_sh.at[dv], sem).wait()

    def _scale(gb, wv):
        def _grp(g2, cc):
            wvec = wv[pl.ds(pl.multiple_of(g2 * 16, 16), 16)]
            for l in range(16):
                wj = wvec[l]
                r = g2 * 16 + l
                for q in range(FCH):
                    sl = pl.ds(q * 16, 16)
                    gb[r, sl] = gb[r, sl] * wj
            return cc

        lax.fori_loop(0, KB // 16, _grp, 0)

    slots = ((gbuf0, dstv0, wv0, semg0, sems0),
             (gbuf1, dstv1, wv1, semg1, sems1),
             (gbuf2, dstv2, wv2, semg2, sems2))
    for k, (gb, dv, wv, sg, ss) in enumerate(slots):
        _fetch_start(gb, dv, wv, k, sg)

    def _zimg_row(i, carry):
        for q in range(FCH):
            zimg[i, pl.ds(q * 16, 16)] = zero16
        return carry

    lax.fori_loop(0, 16, _zimg_row, 0)

    def _zfire(i, carry):
        pltpu.async_copy(zimg, agg_sh.at[pl.ds(zbase + i * 16, 16)], semE)
        return carry

    lax.fori_loop(0, RPT // 16, _zfire, 0)

    @pl.when(s < 2)
    def _zero_extra():
        xb = pl.multiple_of(XBASE + s * 8, 8)
        pltpu.async_copy(zimg.at[pl.ds(0, 8)], agg_sh.at[pl.ds(xb, 8)], semE)

    def _zdrain(i, carry):
        pltpu.make_async_copy(zimg, agg_sh.at[pl.ds(zbase, 16)], semE).wait()
        return carry

    lax.fori_loop(0, RPT // 16, _zdrain, 0)

    @pl.when(s < 2)
    def _zdrain_extra():
        pltpu.make_async_copy(zimg.at[pl.ds(0, 8)],
                              agg_sh.at[pl.ds(zbase, 8)], semE).wait()

    plsc.subcore_barrier()

    def _triple(p, carry):
        a = NSLOT * p
        for k, (gb, dv, wv, sg, ss) in enumerate(slots):
            _fetch_wait(gb, dv, wv, sg)
            _scale(gb, wv)
            _sstart(gb, dv, ss)
        for k, (gb, dv, wv, sg, ss) in enumerate(slots):
            _swait(gb, dv, ss)
            nj = jnp.minimum(a + NSLOT + k, NBT - 1)
            _fetch_start(gb, dv, wv, nj, sg)
        return carry

    lax.fori_loop(0, NTRI, _triple, 0)
    for k in range(2):
        gb, dv, wv, sg, ss = slots[k]
        _fetch_wait(gb, dv, wv, sg)
        _scale(gb, wv)
        _sstart(gb, dv, ss)
    _fetch_wait(slots[2][0], slots[2][1], slots[2][2], slots[2][3])
    for k in range(2):
        gb, dv, wv, sg, ss = slots[k]
        _swait(gb, dv, ss)
    plsc.subcore_barrier()

    pltpu.sync_copy(agg_sh.at[pl.ds(zbase, RPT)],
                    out_hbm.at[pl.ds(zbase, RPT), pl.ds(cH, H)])

    @pl.when(s < 2)
    def _write_extra():
        xb = pl.multiple_of(XBASE + s * 8, 8)
        pltpu.sync_copy(agg_sh.at[pl.ds(xb, 8)],
                        out_hbm.at[pl.ds(xb, 8), pl.ds(cH, H)])


_sc_aggregate = functools.partial(
    pl.kernel,
    out_type=jax.ShapeDtypeStruct((N, D_OUT), jnp.float32),
    mesh=plsc.VectorSubcoreMesh(core_axis_name="c", subcore_axis_name="s"),
    scratch_types=[
        pltpu.VMEM((KB, H), jnp.float32),
        pltpu.VMEM((KB, H), jnp.float32),
        pltpu.VMEM((KB, H), jnp.float32),
        pltpu.VMEM((EPT,), jnp.int32),
        pltpu.VMEM((KB,), jnp.int32),
        pltpu.VMEM((KB,), jnp.int32),
        pltpu.VMEM((KB,), jnp.int32),
        pltpu.VMEM((KB,), jnp.float32),
        pltpu.VMEM((KB,), jnp.float32),
        pltpu.VMEM((KB,), jnp.float32),
        pltpu.VMEM((16, H), jnp.float32),
        pltpu.VMEM_SHARED((N, H), jnp.float32),
        pltpu.SemaphoreType.DMA,
        pltpu.SemaphoreType.DMA,
        pltpu.SemaphoreType.DMA,
        pltpu.SemaphoreType.DMA,
        pltpu.SemaphoreType.DMA,
        pltpu.SemaphoreType.DMA,
        pltpu.SemaphoreType.DMA,
    ],
)(_sc_body)


def _ep_body(a_ref, norm_ref, b_ref, o_ref):
    o_ref[...] = jnp.maximum(a_ref[...] * norm_ref[...] + b_ref[...], 0.0)


def _tc_epilogue(agg, norm, b):
    RB = 2000
    return pl.pallas_call(
        _ep_body,
        grid=(N // RB,),
        in_specs=[
            pl.BlockSpec((RB, D_OUT), lambda r: (r, 0)),
            pl.BlockSpec((RB, 1), lambda r: (r, 0)),
            pl.BlockSpec((1, D_OUT), lambda r: (0, 0)),
        ],
        out_specs=pl.BlockSpec((RB, D_OUT), lambda r: (r, 0)),
        out_shape=jax.ShapeDtypeStruct((N, D_OUT), jnp.float32),
    )(agg, norm, b.reshape(1, D_OUT))


def kernel(h, edge_index, norm, edge_weight, W, b):
    hn = _matmul_halves(h, W, norm)
    agg = _sc_aggregate(hn, edge_index.reshape(2 * E), edge_weight)
    return _tc_epilogue(agg, norm, b)


# speedup vs baseline: 13.1370x; 1.0000x over previous
import functools

import jax
import jax.numpy as jnp
from jax import lax
from jax.experimental import pallas as pl
from jax.experimental.pallas import tpu as pltpu
from jax.experimental.pallas import tpu_sc as plsc

N, E, D_IN, D_OUT = 10000, 160000, 256, 256
H = D_OUT // 2
NTILES = 16
EPT = E // NTILES
KB = 80
NBT = EPT // KB
NSLOT = 3
NTRI = (NBT - 2) // NSLOT
RPT = 624
XBASE = NTILES * RPT
FCH = H // 16


def _mm_body(h_ref, w_ref, norm_ref, o_ref):
    hw = jax.lax.dot_general(
        h_ref[...], w_ref[...], (((1,), (0,)), ((), ())),
        preferred_element_type=jnp.float32)
    o_ref[...] = hw * norm_ref[...]


def _matmul_halves(h, W, norm):
    RB = 2000
    return pl.pallas_call(
        _mm_body,
        grid=(2, N // RB),
        in_specs=[
            pl.BlockSpec((RB, D_IN), lambda c, r: (r, 0)),
            pl.BlockSpec((D_IN, H), lambda c, r: (0, c)),
            pl.BlockSpec((RB, 1), lambda c, r: (r, 0)),
        ],
        out_specs=pl.BlockSpec((RB, H), lambda c, r: (c * (N // RB) + r, 0)),
        out_shape=jax.ShapeDtypeStruct((2 * N, H), jnp.float32),
    )(h, W, norm)


def _sc_body(hn_hbm, ei_hbm, ew_hbm, out_hbm,
             gbuf0, gbuf1, gbuf2, srcC, dstv0, dstv1, dstv2,
             wv0, wv1, wv2, zimg, agg_sh,
             semg0, semg1, semg2, sems0, sems1, sems2, semE):
    c = lax.axis_index("c")
    s = lax.axis_index("s")
    zbase = pl.multiple_of(s * RPT, 8)
    cH = pl.multiple_of(c * H, 8)
    ebase = pl.multiple_of(s * EPT, 8)
    zero16 = jnp.zeros((16,), jnp.float32)

    pltpu.sync_copy(ei_hbm.at[pl.ds(ebase, EPT)], srcC)
    cN = jnp.full((16,), c * N, jnp.int32)

    def _off_row(i, carry):
        sl = pl.ds(pl.multiple_of(i * 16, 16), 16)
        srcC[sl] = srcC[sl] + cN
        return carry

    lax.fori_loop(0, EPT // 16, _off_row, 0)

    def _fetch_start(gb, dv, wv, j, sem):
        pltpu.async_copy(
            ei_hbm.at[pl.ds(pl.multiple_of(E + ebase + j * KB, 8), KB)],
            dv, sem)
        pltpu.async_copy(
            ew_hbm.at[pl.ds(pl.multiple_of(ebase + j * KB, 8), KB)], wv, sem)
        pltpu.async_copy(
            hn_hbm.at[srcC.at[pl.ds(pl.multiple_of(j * KB, 16), KB)]],
            gb, sem)

    def _fetch_wait(gb, dv, wv, sem):
        pltpu.make_async_copy(ei_hbm.at[pl.ds(E, KB)], dv, sem).wait()
        pltpu.make_async_copy(ew_hbm.at[pl.ds(0, KB)], wv, sem).wait()
        pltpu.make_async_copy(
            hn_hbm.at[srcC.at[pl.ds(0, KB)]], gb, sem).wait()

    def _sstart(gb, dv, sem):
        pltpu.make_async_copy(gb, agg_sh.at[dv], sem).start(add=True)

    def _swait(gb, dv, sem):
        pltpu.make_async_copy(gb, agg_sh.at[dv], sem).wait()

    def _scale(gb, wv):
        def _grp(g2, cc):
            wvec = wv[pl.ds(pl.multiple_of(g2 * 16, 16), 16)]
            for l in range(16):
                wj = wvec[l]
                r = g2 * 16 + l
                for q in range(FCH):
                    sl = pl.ds(q * 16, 16)
                    gb[r, sl] = gb[r, sl] * wj
            return cc

        lax.fori_loop(0, KB // 16, _grp, 0)

    slots = ((gbuf0, dstv0, wv0, semg0, sems0),
             (gbuf1, dstv1, wv1, semg1, sems1),
             (gbuf2, dstv2, wv2, semg2, sems2))
    for k, (gb, dv, wv, sg, ss) in enumerate(slots):
        _fetch_start(gb, dv, wv, k, sg)

    def _zimg_row(i, carry):
        for q in range(FCH):
            zimg[i, pl.ds(q * 16, 16)] = zero16
        return carry

    lax.fori_loop(0, 16, _zimg_row, 0)

    def _zfire(i, carry):
        pltpu.async_copy(zimg, agg_sh.at[pl.ds(zbase + i * 16, 16)], semE)
        return carry

    lax.fori_loop(0, RPT // 16, _zfire, 0)

    @pl.when(s < 2)
    def _zero_extra():
        xb = pl.multiple_of(XBASE + s * 8, 8)
        pltpu.async_copy(zimg.at[pl.ds(0, 8)], agg_sh.at[pl.ds(xb, 8)], semE)

    def _zdrain(i, carry):
        pltpu.make_async_copy(zimg, agg_sh.at[pl.ds(zbase, 16)], semE).wait()
        return carry

    lax.fori_loop(0, RPT // 16, _zdrain, 0)

    @pl.when(s < 2)
    def _zdrain_extra():
        pltpu.make_async_copy(zimg.at[pl.ds(0, 8)],
                              agg_sh.at[pl.ds(zbase, 8)], semE).wait()

    plsc.subcore_barrier()

    def _triple(p, carry):
        a = NSLOT * p
        for k, (gb, dv, wv, sg, ss) in enumerate(slots):
            _fetch_wait(gb, dv, wv, sg)
            _scale(gb, wv)
            _sstart(gb, dv, ss)
        for k, (gb, dv, wv, sg, ss) in enumerate(slots):
            _swait(gb, dv, ss)
            nj = jnp.minimum(a + NSLOT + k, NBT - 1)
            _fetch_start(gb, dv, wv, nj, sg)
        return carry

    lax.fori_loop(0, NTRI, _triple, 0)
    for k in range(2):
        gb, dv, wv, sg, ss = slots[k]
        _fetch_wait(gb, dv, wv, sg)
        _scale(gb, wv)
        _sstart(gb, dv, ss)
    _fetch_wait(slots[2][0], slots[2][1], slots[2][2], slots[2][3])
    for k in range(2):
        gb, dv, wv, sg, ss = slots[k]
        _swait(gb, dv, ss)
    plsc.subcore_barrier()

    pltpu.sync_copy(agg_sh.at[pl.ds(zbase, RPT)],
                    out_hbm.at[pl.ds(zbase, RPT), pl.ds(cH, H)])

    @pl.when(s < 2)
    def _write_extra():
        xb = pl.multiple_of(XBASE + s * 8, 8)
        pltpu.sync_copy(agg_sh.at[pl.ds(xb, 8)],
                        out_hbm.at[pl.ds(xb, 8), pl.ds(cH, H)])


_sc_aggregate = functools.partial(
    pl.kernel,
    out_type=jax.ShapeDtypeStruct((N, D_OUT), jnp.float32),
    mesh=plsc.VectorSubcoreMesh(core_axis_name="c", subcore_axis_name="s"),
    scratch_types=[
        pltpu.VMEM((KB, H), jnp.float32),
        pltpu.VMEM((KB, H), jnp.float32),
        pltpu.VMEM((KB, H), jnp.float32),
        pltpu.VMEM((EPT,), jnp.int32),
        pltpu.VMEM((KB,), jnp.int32),
        pltpu.VMEM((KB,), jnp.int32),
        pltpu.VMEM((KB,), jnp.int32),
        pltpu.VMEM((KB,), jnp.float32),
        pltpu.VMEM((KB,), jnp.float32),
        pltpu.VMEM((KB,), jnp.float32),
        pltpu.VMEM((16, H), jnp.float32),
        pltpu.VMEM_SHARED((N, H), jnp.float32),
        pltpu.SemaphoreType.DMA,
        pltpu.SemaphoreType.DMA,
        pltpu.SemaphoreType.DMA,
        pltpu.SemaphoreType.DMA,
        pltpu.SemaphoreType.DMA,
        pltpu.SemaphoreType.DMA,
        pltpu.SemaphoreType.DMA,
    ],
)(_sc_body)


def _ep_body(a_ref, norm_ref, b_ref, o_ref):
    o_ref[...] = jnp.maximum(a_ref[...] * norm_ref[...] + b_ref[...], 0.0)


def _tc_epilogue(agg, norm, b):
    RB = 2000
    return pl.pallas_call(
        _ep_body,
        grid=(N // RB,),
        in_specs=[
            pl.BlockSpec((RB, D_OUT), lambda r: (r, 0)),
            pl.BlockSpec((RB, 1), lambda r: (r, 0)),
            pl.BlockSpec((1, D_OUT), lambda r: (0, 0)),
        ],
        out_specs=pl.BlockSpec((RB, D_OUT), lambda r: (r, 0)),
        out_shape=jax.ShapeDtypeStruct((N, D_OUT), jnp.float32),
    )(agg, norm, b.reshape(1, D_OUT))


def kernel(h, edge_index, norm, edge_weight, W, b):
    hn = _matmul_halves(h, W, norm)
    agg = _sc_aggregate(hn, edge_index.reshape(2 * E), edge_weight)
    return _tc_epilogue(agg, norm, b)
